# Initial kernel scaffold; baseline (speedup 1.0000x reference)
import jax
import jax.numpy as jnp
from jax.experimental import pallas as pl


def kernel(x, is_training, test_local_stats, binary_set):
    raise NotImplementedError("write your pallas kernel here")


# speedup vs baseline: 6.8920x; 6.8920x over previous
import functools

import jax
import jax.numpy as jnp
from jax import lax
from jax.experimental import pallas as pl
from jax.experimental.pallas import tpu as pltpu
from jax.experimental.pallas import tpu_sc as plsc

N = 8388608
TW = 2 ** 19
NC, NS, L = 2, 16, 16
NW = NC * NS
KPT = N // NW
CHUNK = 8192
NCHUNK = KPT // CHUNK
GROUPS = CHUNK // 64
TAB_SLICE = TW // NS

_mesh = plsc.VectorSubcoreMesh(
    core_axis_name="c", subcore_axis_name="s", num_cores=NC, num_subcores=NS)


@functools.partial(
    pl.kernel,
    out_type=jax.ShapeDtypeStruct((N // 4,), jnp.int32),
    mesh=_mesh,
    scratch_types=[
        pltpu.VMEM_SHARED((TW,), jnp.int32),
        pltpu.VMEM((2 * CHUNK,), jnp.int32),
        pltpu.VMEM((CHUNK,), jnp.int32),
        pltpu.VMEM((CHUNK,), jnp.int32),
        pltpu.VMEM((CHUNK,), jnp.int32),
        pltpu.VMEM((CHUNK // 4,), jnp.int32),
    ],
    compiler_params=pltpu.CompilerParams(needs_layout_passes=False),
)
def _lcg_sc(x_hbm, tab_hbm, out_hbm, tab_sh, xbuf, wqbuf, bpbuf, twbuf, obuf):
    cid = lax.axis_index("c")
    sid = lax.axis_index("s")
    wid = sid * NC + cid

    toff = sid * jnp.int32(TAB_SLICE)
    pltpu.sync_copy(tab_hbm.at[pl.ds(toff, TAB_SLICE)],
                    tab_sh.at[pl.ds(toff, TAB_SLICE)])
    plsc.subcore_barrier()

    lanes8 = lax.iota(jnp.int32, L) * 8
    c7 = jnp.full((L,), 7, jnp.int32)
    c5 = jnp.full((L,), 5, jnp.int32)
    c31 = jnp.full((L,), 31, jnp.int32)
    c1 = jnp.full((L,), 1, jnp.int32)

    def chunk_body(g, carry):
        xoff = wid * jnp.int32(2 * KPT) + g * jnp.int32(2 * CHUNK)
        pltpu.sync_copy(x_hbm.at[pl.ds(xoff, 2 * CHUNK)], xbuf)

        def idx_body(t, c2):
            o = t * jnp.int32(64)
            off = t * jnp.int32(128)
            for j in range(4):
                hi = plsc.load_gather(xbuf, [lanes8 + (off + jnp.int32(2 * j + 1))])
                i24 = lax.shift_right_logical(hi, c7)
                wqbuf[pl.ds(o + jnp.int32(j * 16), L)] = lax.shift_right_logical(i24, c5)
                bpbuf[pl.ds(o + jnp.int32(j * 16), L)] = lax.bitwise_and(i24, c31)
            return c2
        lax.fori_loop(jnp.int32(0), jnp.int32(GROUPS), idx_body, jnp.int32(0), unroll=False)

        pltpu.sync_copy(tab_sh.at[wqbuf], twbuf)

        def pack_body(t, c2):
            o = t * jnp.int32(64)
            acc = jnp.zeros((L,), jnp.int32)
            for j in range(4):
                tw = twbuf[pl.ds(o + jnp.int32(j * 16), L)]
                bp = bpbuf[pl.ds(o + jnp.int32(j * 16), L)]
                bit = lax.bitwise_and(lax.shift_right_logical(tw, bp), c1)
                if j:
                    bit = lax.shift_left(bit, jnp.full((L,), 8 * j, jnp.int32))
                acc = lax.bitwise_or(acc, bit)
            obuf[pl.ds(t * jnp.int32(16), L)] = acc
            return c2
        lax.fori_loop(jnp.int32(0), jnp.int32(GROUPS), pack_body, jnp.int32(0), unroll=False)

        ooff = wid * jnp.int32(KPT // 4) + g * jnp.int32(CHUNK // 4)
        pltpu.sync_copy(obuf, out_hbm.at[pl.ds(ooff, CHUNK // 4)])
        return carry

    lax.fori_loop(jnp.int32(0), jnp.int32(NCHUNK), chunk_body, jnp.int32(0), unroll=False)


def kernel(x, is_training, test_local_stats, binary_set):
    x32 = lax.bitcast_convert_type(x, jnp.int32).reshape(2 * N)
    tab = lax.bitcast_convert_type(binary_set.reshape(TW, 4), jnp.int32)
    outw = _lcg_sc(x32, tab)
    return lax.bitcast_convert_type(outw, jnp.uint8).reshape(N).astype(jnp.bool_)


# speedup vs baseline: 28.8363x; 4.1840x over previous
import functools

import jax
import jax.numpy as jnp
from jax import lax
from jax.experimental import pallas as pl
from jax.experimental.pallas import tpu as pltpu
from jax.experimental.pallas import tpu_sc as plsc

N = 8388608
TW = 2 ** 19
NC, NS, L = 2, 16, 16
NW = NC * NS
KPT = N // NW
CHUNK = 8192
NCHUNK = KPT // CHUNK
GROUPS = CHUNK // 64
TAB_SLICE = TW // NS

_mesh = plsc.VectorSubcoreMesh(
    core_axis_name="c", subcore_axis_name="s", num_cores=NC, num_subcores=NS)


@functools.partial(
    pl.kernel,
    out_type=jax.ShapeDtypeStruct((N // 4,), jnp.int32),
    mesh=_mesh,
    scratch_types=[
        pltpu.VMEM_SHARED((TW,), jnp.int32),
        pltpu.VMEM((2 * CHUNK,), jnp.int32),
        pltpu.VMEM((2 * CHUNK // 128, 128), jnp.int32),
        pltpu.VMEM((CHUNK,), jnp.int32),
        pltpu.VMEM((CHUNK,), jnp.int32),
        pltpu.VMEM((CHUNK,), jnp.int32),
        pltpu.VMEM((CHUNK // 4,), jnp.int32),
    ],
    compiler_params=pltpu.CompilerParams(needs_layout_passes=False),
)
def _lcg_sc(x_hbm, tab_hbm, out8_hbm, tab_sh, sbuf, xbuf, wqbuf, bpbuf, twbuf, obuf):
    cid = lax.axis_index("c")
    sid = lax.axis_index("s")
    wid = sid * NC + cid

    for st in range(TAB_SLICE // (2 * CHUNK)):
        woff = pl.multiple_of(sid * jnp.int32(TAB_SLICE) + jnp.int32(st * 2 * CHUNK), 8)
        pltpu.sync_copy(tab_hbm.at[pl.ds(woff, 2 * CHUNK)], sbuf)
        pltpu.sync_copy(sbuf, tab_sh.at[pl.ds(woff, 2 * CHUNK)])
    plsc.subcore_barrier()

    lanes8 = lax.iota(jnp.int32, L) * 8
    zero16 = jnp.zeros((L,), jnp.int32)
    c7 = jnp.full((L,), 7, jnp.int32)
    c5 = jnp.full((L,), 5, jnp.int32)
    c31 = jnp.full((L,), 31, jnp.int32)
    c1 = jnp.full((L,), 1, jnp.int32)

    def chunk_body(g, carry):
        kbase = wid * jnp.int32(KPT) + g * jnp.int32(CHUNK)
        xrow = pl.multiple_of((kbase * jnp.int32(2)) // jnp.int32(128), 8)
        pltpu.sync_copy(x_hbm.at[pl.ds(xrow, 2 * CHUNK // 128), :], xbuf)

        def idx_body(t, c2):
            o = t * jnp.int32(64)
            rvec = zero16 + t
            for j in range(4):
                hi = plsc.load_gather(xbuf, [rvec, lanes8 + jnp.int32(2 * j + 1)])
                i24 = lax.shift_right_logical(hi, c7)
                wqbuf[pl.ds(o + jnp.int32(j * 16), L)] = lax.shift_right_logical(i24, c5)
                bpbuf[pl.ds(o + jnp.int32(j * 16), L)] = lax.bitwise_and(i24, c31)
            return c2
        lax.fori_loop(jnp.int32(0), jnp.int32(GROUPS), idx_body, jnp.int32(0), unroll=False)

        pltpu.sync_copy(tab_sh.at[wqbuf], twbuf)

        def pack_body(t, c2):
            o = t * jnp.int32(64)
            acc = jnp.zeros((L,), jnp.int32)
            for j in range(4):
                tw = twbuf[pl.ds(o + jnp.int32(j * 16), L)]
                bp = bpbuf[pl.ds(o + jnp.int32(j * 16), L)]
                bit = lax.bitwise_and(lax.shift_right_logical(tw, bp), c1)
                if j:
                    bit = lax.shift_left(bit, jnp.full((L,), 8 * j, jnp.int32))
                acc = lax.bitwise_or(acc, bit)
            obuf[pl.ds(t * jnp.int32(16), L)] = acc
            return c2
        lax.fori_loop(jnp.int32(0), jnp.int32(GROUPS), pack_body, jnp.int32(0), unroll=False)

        obase = pl.multiple_of(kbase // jnp.int32(4), 8)
        pltpu.sync_copy(obuf, out8_hbm.at[pl.ds(obase, CHUNK // 4)])
        return carry

    lax.fori_loop(jnp.int32(0), jnp.int32(NCHUNK), chunk_body, jnp.int32(0), unroll=False)


def kernel(x, is_training, test_local_stats, binary_set):
    x32 = lax.bitcast_convert_type(x, jnp.int32).reshape(2 * N // 128, 128)
    tab = lax.bitcast_convert_type(binary_set.reshape(TW, 4), jnp.int32)
    outw = _lcg_sc(x32, tab)
    return lax.bitcast_convert_type(outw, jnp.uint8).reshape(N).astype(jnp.bool_)


# speedup vs baseline: 28.9176x; 1.0028x over previous
import functools

import jax
import jax.numpy as jnp
from jax import lax
from jax.experimental import pallas as pl
from jax.experimental.pallas import tpu as pltpu
from jax.experimental.pallas import tpu_sc as plsc

N = 8388608
TW = 2 ** 19
NC, NS, L = 2, 16, 16
NW = NC * NS
KPT = N // NW
CHUNK = 8192
NCHUNK = KPT // CHUNK
GROUPS = CHUNK // 64
TAB_SLICE = TW // NS

_mesh = plsc.VectorSubcoreMesh(
    core_axis_name="c", subcore_axis_name="s", num_cores=NC, num_subcores=NS)


@functools.partial(
    pl.kernel,
    out_type=jax.ShapeDtypeStruct((N // 4 // 128, 128), jnp.int32),
    mesh=_mesh,
    scratch_types=[
        pltpu.VMEM_SHARED((TW,), jnp.int32),
        pltpu.VMEM((2 * CHUNK,), jnp.int32),
        pltpu.VMEM((2 * CHUNK // 128, 128), jnp.int32),
        pltpu.VMEM((CHUNK,), jnp.int32),
        pltpu.VMEM((CHUNK,), jnp.int32),
        pltpu.VMEM((CHUNK,), jnp.int32),
        pltpu.VMEM((CHUNK // 4 // 128, 128), jnp.int32),
    ],
    compiler_params=pltpu.CompilerParams(
        needs_layout_passes=False, use_tc_tiling_on_sc=True),
)
def _lcg_sc(x_hbm, tab_hbm, out8_hbm, tab_sh, sbuf, xbuf, wqbuf, bpbuf, twbuf, obuf):
    cid = lax.axis_index("c")
    sid = lax.axis_index("s")
    wid = sid * NC + cid

    for st in range(TAB_SLICE // (2 * CHUNK)):
        woff = pl.multiple_of(sid * jnp.int32(TAB_SLICE) + jnp.int32(st * 2 * CHUNK), 8)
        pltpu.sync_copy(tab_hbm.at[pl.ds(woff, 2 * CHUNK)], sbuf)
        pltpu.sync_copy(sbuf, tab_sh.at[pl.ds(woff, 2 * CHUNK)])
    plsc.subcore_barrier()

    lanes8 = lax.iota(jnp.int32, L) * 8
    zero16 = jnp.zeros((L,), jnp.int32)
    c7 = jnp.full((L,), 7, jnp.int32)
    c5 = jnp.full((L,), 5, jnp.int32)
    c31 = jnp.full((L,), 31, jnp.int32)
    c1 = jnp.full((L,), 1, jnp.int32)

    def chunk_body(g, carry):
        kbase = wid * jnp.int32(KPT) + g * jnp.int32(CHUNK)
        xrow = pl.multiple_of((kbase * jnp.int32(2)) // jnp.int32(128), 8)
        pltpu.sync_copy(x_hbm.at[pl.ds(xrow, 2 * CHUNK // 128), :], xbuf)

        def idx_body(t, c2):
            o = t * jnp.int32(64)
            rvec = zero16 + t
            for j in range(4):
                hi = plsc.load_gather(xbuf, [rvec, lanes8 + jnp.int32(2 * j + 1)])
                i24 = lax.shift_right_logical(hi, c7)
                wqbuf[pl.ds(o + jnp.int32(j * 16), L)] = lax.shift_right_logical(i24, c5)
                bpbuf[pl.ds(o + jnp.int32(j * 16), L)] = lax.bitwise_and(i24, c31)
            return c2
        lax.fori_loop(jnp.int32(0), jnp.int32(GROUPS), idx_body, jnp.int32(0), unroll=False)

        pltpu.sync_copy(tab_sh.at[wqbuf], twbuf)

        def pack_body(t, c2):
            o = t * jnp.int32(64)
            acc = jnp.zeros((L,), jnp.int32)
            for j in range(4):
                tw = twbuf[pl.ds(o + jnp.int32(j * 16), L)]
                bp = bpbuf[pl.ds(o + jnp.int32(j * 16), L)]
                bit = lax.bitwise_and(lax.shift_right_logical(tw, bp), c1)
                if j:
                    bit = lax.shift_left(bit, jnp.full((L,), 8 * j, jnp.int32))
                acc = lax.bitwise_or(acc, bit)
            obuf[t >> 3, pl.ds((t & jnp.int32(7)) * jnp.int32(16), L)] = acc
            return c2
        lax.fori_loop(jnp.int32(0), jnp.int32(GROUPS), pack_body, jnp.int32(0), unroll=False)

        orow = pl.multiple_of(kbase // jnp.int32(4 * 128), 8)
        pltpu.sync_copy(obuf, out8_hbm.at[pl.ds(orow, CHUNK // 4 // 128), :])
        return carry

    lax.fori_loop(jnp.int32(0), jnp.int32(NCHUNK), chunk_body, jnp.int32(0), unroll=False)


def kernel(x, is_training, test_local_stats, binary_set):
    x32 = lax.bitcast_convert_type(x, jnp.int32).reshape(2 * N // 128, 128)
    tab = lax.bitcast_convert_type(binary_set.reshape(TW, 4), jnp.int32)
    outw = _lcg_sc(x32, tab)
    outb = lax.bitcast_convert_type(outw, jnp.uint8)
    return outb.astype(jnp.bool_).reshape(N)


# speedup vs baseline: 38.4944x; 1.3312x over previous
import functools

import jax
import jax.numpy as jnp
from jax import lax
from jax.experimental import pallas as pl
from jax.experimental.pallas import tpu as pltpu
from jax.experimental.pallas import tpu_sc as plsc

N = 8388608
TW = 2 ** 19
NC, NS, L = 2, 16, 16
NW = NC * NS
KPT = N // NW
CHUNK = 16384
NCHUNK = KPT // CHUNK
GROUPS = CHUNK // 64
TAB_SLICE = TW // NS

_mesh = plsc.VectorSubcoreMesh(
    core_axis_name="c", subcore_axis_name="s", num_cores=NC, num_subcores=NS)


@functools.partial(
    pl.kernel,
    out_type=jax.ShapeDtypeStruct((N // 4 // 128, 128), jnp.int32),
    mesh=_mesh,
    scratch_types=[
        pltpu.VMEM_SHARED((TW,), jnp.int32),
        pltpu.VMEM((CHUNK,), jnp.int32),
        pltpu.VMEM((CHUNK,), jnp.int32),
        pltpu.VMEM((CHUNK,), jnp.int32),
        pltpu.VMEM((CHUNK,), jnp.int32),
        pltpu.VMEM((CHUNK // 4 // 128, 128), jnp.int32),
    ],
    compiler_params=pltpu.CompilerParams(
        needs_layout_passes=False, use_tc_tiling_on_sc=True),
)
def _lcg_sc(x_hbm, tab_hbm, out8_hbm, tab_sh, xbuf, wqbuf, bpbuf, twbuf, obuf):
    cid = lax.axis_index("c")
    sid = lax.axis_index("s")
    wid = sid * NC + cid


    for st in range(TAB_SLICE // CHUNK):
        woff = pl.multiple_of(sid * jnp.int32(TAB_SLICE) + jnp.int32(st * CHUNK), 8)
        pltpu.sync_copy(tab_hbm.at[pl.ds(woff, CHUNK)], xbuf)
        pltpu.sync_copy(xbuf, tab_sh.at[pl.ds(woff, CHUNK)])
    plsc.subcore_barrier()

    lanes4 = lax.iota(jnp.int32, L) * 4
    c7 = jnp.full((L,), 7, jnp.int32)
    c5 = jnp.full((L,), 5, jnp.int32)
    c31 = jnp.full((L,), 31, jnp.int32)
    c1 = jnp.full((L,), 1, jnp.int32)

    def chunk_body(g, carry):
        kbase = pl.multiple_of(wid * jnp.int32(KPT) + g * jnp.int32(CHUNK), 8)
        pltpu.sync_copy(xv.at[1, pl.ds(kbase, CHUNK)], xbuf)

        def idx_body(t, c2):
            o = t * jnp.int32(64)
            for j in range(4):
                hi = plsc.load_gather(xbuf, [lanes4 + (o + jnp.int32(j))])
                i24 = lax.shift_right_logical(hi, c7)
                wqbuf[pl.ds(o + jnp.int32(j * 16), L)] = lax.shift_right_logical(i24, c5)
                bpbuf[pl.ds(o + jnp.int32(j * 16), L)] = lax.bitwise_and(i24, c31)
            return c2
        lax.fori_loop(jnp.int32(0), jnp.int32(GROUPS), idx_body, jnp.int32(0), unroll=False)

        pltpu.sync_copy(tab_sh.at[wqbuf], twbuf)

        def pack_body(t, c2):
            o = t * jnp.int32(64)
            acc = jnp.zeros((L,), jnp.int32)
            for j in range(4):
                tw = twbuf[pl.ds(o + jnp.int32(j * 16), L)]
                bp = bpbuf[pl.ds(o + jnp.int32(j * 16), L)]
                bit = lax.bitwise_and(lax.shift_right_logical(tw, bp), c1)
                if j:
                    bit = lax.shift_left(bit, jnp.full((L,), 8 * j, jnp.int32))
                acc = lax.bitwise_or(acc, bit)
            obuf[t >> 3, pl.ds((t & jnp.int32(7)) * jnp.int32(16), L)] = acc
            return c2
        lax.fori_loop(jnp.int32(0), jnp.int32(GROUPS), pack_body, jnp.int32(0), unroll=False)

        orow = pl.multiple_of(kbase // jnp.int32(4 * 128), 8)
        pltpu.sync_copy(obuf, out8_hbm.at[pl.ds(orow, CHUNK // 4 // 128), :])
        return carry

    lax.fori_loop(jnp.int32(0), jnp.int32(NCHUNK), chunk_body, jnp.int32(0), unroll=False)


def kernel(x, is_training, test_local_stats, binary_set):
    xhi = lax.shift_right_logical(x, 32).astype(jnp.int32)
    x32 = xhi.reshape(N // CHUNK, CHUNK)
    tab = lax.bitcast_convert_type(
        binary_set.reshape(TW // 128, 128, 4), jnp.int32).reshape(TW)
    outw = _lcg_sc(x32, tab)
    outb = lax.bitcast_convert_type(outw, jnp.uint8)
    return outb.reshape(N).astype(jnp.bool_)
